# Initial kernel scaffold; baseline (speedup 1.0000x reference)
import jax
import jax.numpy as jnp
from jax.experimental import pallas as pl


def kernel(obs, action, edge_index, W1, b1, W2, b2, Wq1a, bq1a, Wq1b, bq1b, Wq2a, bq2a, Wq2b, bq2b):
    raise NotImplementedError("write your pallas kernel here")


# speedup vs baseline: 8.8213x; 8.8213x over previous
import functools

import jax
import jax.numpy as jnp
from jax import lax
from jax.experimental import pallas as pl
from jax.experimental.pallas import tpu as pltpu
from jax.experimental.pallas import tpu_sc as plsc

N = 10000
NPAD = 10240
D = 128
E = 320000
EPAD = 323584
NC, NS = 2, 16
NW = NC * NS
EPT = EPAD // NW
CHUNK = 128
NCHUNK = EPT // CHUNK
RPT = NPAD // NS
BM = 256


def _sc_mesh():
    return plsc.VectorSubcoreMesh(
        core_axis_name="c", subcore_axis_name="s",
        num_cores=NC, num_subcores=NS)


def _fill_rows(ref, nrows, value):
    vec = jnp.full((16,), value, jnp.float32)

    def fill(r, carry):
        for q in range(D // 16):
            ref[r, pl.ds(q * 16, 16)] = vec
        return carry
    lax.fori_loop(0, nrows, fill, 0)


def _sc_degree(dst_pad):
    @functools.partial(
        pl.kernel,
        out_type=jax.ShapeDtypeStruct((NC, NPAD, D), jnp.float32),
        mesh=_sc_mesh(),
        scratch_types=[
            pltpu.VMEM((CHUNK,), jnp.int32),
            pltpu.VMEM((CHUNK, D), jnp.float32),
            pltpu.VMEM_SHARED((NPAD, D), jnp.float32),
        ],
    )
    def deg_kernel(dst_hbm, out_hbm, didx, ones, deg_sh):
        cid = lax.axis_index("c")
        sid = lax.axis_index("s")
        wid = cid * NS + sid

        _fill_rows(ones, CHUNK, 0.0)

        def zero_slab(k, carry):
            pltpu.sync_copy(ones, deg_sh.at[pl.ds(sid * RPT + k * CHUNK, CHUNK)])
            return carry
        lax.fori_loop(0, RPT // CHUNK, zero_slab, 0)
        _fill_rows(ones, CHUNK, 1.0)
        plsc.subcore_barrier()

        def step(j, carry):
            base = wid * EPT + j * CHUNK
            pltpu.sync_copy(dst_hbm.at[pl.ds(base, CHUNK)], didx)
            pltpu.sync_copy(ones, deg_sh.at[didx], add=True)
            return carry
        lax.fori_loop(0, NCHUNK, step, 0)
        plsc.subcore_barrier()

        pltpu.sync_copy(deg_sh.at[pl.ds(sid * RPT, RPT)],
                        out_hbm.at[cid, pl.ds(sid * RPT, RPT)])

    return deg_kernel(dst_pad)


def _sc_aggregate(m, src_pad, dst_pad):
    @functools.partial(
        pl.kernel,
        out_type=jax.ShapeDtypeStruct((NC, NPAD, D), jnp.float32),
        mesh=_sc_mesh(),
        scratch_types=[
            pltpu.VMEM((CHUNK,), jnp.int32),
            pltpu.VMEM((CHUNK,), jnp.int32),
            pltpu.VMEM((CHUNK, D), jnp.float32),
            pltpu.VMEM((CHUNK, D), jnp.float32),
            pltpu.SemaphoreType.DMA,
            pltpu.VMEM_SHARED((NPAD, D), jnp.float32),
        ],
    )
    def agg_kernel(m_hbm, src_hbm, dst_hbm, out_hbm,
                   sidx, didx, rows, zbuf, gsem, acc_sh):
        cid = lax.axis_index("c")
        sid = lax.axis_index("s")
        wid = cid * NS + sid

        _fill_rows(zbuf, CHUNK, 0.0)

        def zero_slab(k, carry):
            pltpu.sync_copy(zbuf, acc_sh.at[pl.ds(sid * RPT + k * CHUNK, CHUNK)])
            return carry
        lax.fori_loop(0, RPT // CHUNK, zero_slab, 0)
        plsc.subcore_barrier()

        def step(j, carry):
            base = wid * EPT + j * CHUNK
            pltpu.sync_copy(src_hbm.at[pl.ds(base, CHUNK)], sidx)
            pltpu.sync_copy(dst_hbm.at[pl.ds(base, CHUNK)], didx)
            pltpu.async_copy(m_hbm.at[sidx], rows, gsem).wait()
            pltpu.sync_copy(rows, acc_sh.at[didx], add=True)
            return carry
        lax.fori_loop(0, NCHUNK, step, 0)
        plsc.subcore_barrier()

        pltpu.sync_copy(acc_sh.at[pl.ds(sid * RPT, RPT)],
                        out_hbm.at[cid, pl.ds(sid * RPT, RPT)])

    return agg_kernel(m, src_pad, dst_pad)


def _dinv_block(d0_ref, d1_ref, block_idx):
    deg = d0_ref[:, 0:1] + d1_ref[:, 0:1] + 1.0
    dinv = lax.rsqrt(deg)
    row = block_idx * BM + lax.broadcasted_iota(jnp.int32, (BM, 1), 0)
    return jnp.where(row < N, dinv, 0.0)


def _mm_body(x_ref, w_ref, o_ref):
    o_ref[...] = jnp.dot(x_ref[...], w_ref[...],
                         preferred_element_type=jnp.float32)


def _tc_matmul(x, wT):
    return pl.pallas_call(
        _mm_body,
        grid=(NPAD // BM,),
        in_specs=[pl.BlockSpec((BM, D), lambda i: (i, 0)),
                  pl.BlockSpec((D, D), lambda i: (0, 0))],
        out_specs=pl.BlockSpec((BM, D), lambda i: (i, 0)),
        out_shape=jax.ShapeDtypeStruct((NPAD, D), jnp.float32),
    )(x, wT)


def _scale_body(h_ref, d0_ref, d1_ref, o_ref):
    dinv = _dinv_block(d0_ref, d1_ref, pl.program_id(0))
    o_ref[...] = h_ref[...] * dinv


def _tc_scale(h, dg0, dg1):
    return pl.pallas_call(
        _scale_body,
        grid=(NPAD // BM,),
        in_specs=[pl.BlockSpec((BM, D), lambda i: (i, 0)),
                  pl.BlockSpec((BM, D), lambda i: (i, 0)),
                  pl.BlockSpec((BM, D), lambda i: (i, 0))],
        out_specs=pl.BlockSpec((BM, D), lambda i: (i, 0)),
        out_shape=jax.ShapeDtypeStruct((NPAD, D), jnp.float32),
    )(h, dg0, dg1)


def _mid_body(m_ref, a0_ref, a1_ref, d0_ref, d1_ref, b_ref, w_ref, o_ref):
    dinv = _dinv_block(d0_ref, d1_ref, pl.program_id(0))
    x1 = jnp.maximum(
        dinv * (a0_ref[...] + a1_ref[...] + m_ref[...]) + b_ref[...], 0.0)
    o_ref[...] = jnp.dot(x1, w_ref[...],
                         preferred_element_type=jnp.float32) * dinv


def _tc_mid(m1, a0, a1, dg0, dg1, b1, w2T):
    return pl.pallas_call(
        _mid_body,
        grid=(NPAD // BM,),
        in_specs=[pl.BlockSpec((BM, D), lambda i: (i, 0)),
                  pl.BlockSpec((BM, D), lambda i: (i, 0)),
                  pl.BlockSpec((BM, D), lambda i: (i, 0)),
                  pl.BlockSpec((BM, D), lambda i: (i, 0)),
                  pl.BlockSpec((BM, D), lambda i: (i, 0)),
                  pl.BlockSpec((1, D), lambda i: (0, 0)),
                  pl.BlockSpec((D, D), lambda i: (0, 0))],
        out_specs=pl.BlockSpec((BM, D), lambda i: (i, 0)),
        out_shape=jax.ShapeDtypeStruct((NPAD, D), jnp.float32),
    )(m1, a0, a1, dg0, dg1, b1, w2T)


def _final_body(m_ref, a0_ref, a1_ref, d0_ref, d1_ref, b2_ref,
                wq1a_ref, bq1a_ref, wq1b_ref, bq1b_ref,
                wq2a_ref, bq2a_ref, wq2b_ref, bq2b_ref,
                q1_ref, q2_ref):
    dinv = _dinv_block(d0_ref, d1_ref, pl.program_id(0))
    x2 = jnp.maximum(
        dinv * (a0_ref[...] + a1_ref[...] + m_ref[...]) + b2_ref[...], 0.0)
    h1 = jnp.maximum(
        jnp.dot(x2, wq1a_ref[...], preferred_element_type=jnp.float32)
        + bq1a_ref[...], 0.0)
    q1_ref[...] = jnp.dot(h1, wq1b_ref[...],
                          preferred_element_type=jnp.float32) + bq1b_ref[0, 0]
    h2 = jnp.maximum(
        jnp.dot(x2, wq2a_ref[...], preferred_element_type=jnp.float32)
        + bq2a_ref[...], 0.0)
    q2_ref[...] = jnp.dot(h2, wq2b_ref[...],
                          preferred_element_type=jnp.float32) + bq2b_ref[0, 0]


def _tc_final(m2, a0, a1, dg0, dg1, b2,
              wq1aT, bq1a, wq1bT, bq1b, wq2aT, bq2a, wq2bT, bq2b):
    full = lambda r, c: pl.BlockSpec((r, c), lambda i: (0, 0))
    blk = lambda c: pl.BlockSpec((BM, c), lambda i: (i, 0))
    return pl.pallas_call(
        _final_body,
        grid=(NPAD // BM,),
        in_specs=[blk(D), blk(D), blk(D), blk(D), blk(D), full(1, D),
                  full(D, D), full(1, D), full(D, 1), full(1, 1),
                  full(D, D), full(1, D), full(D, 1), full(1, 1)],
        out_specs=[pl.BlockSpec((BM, 1), lambda i: (i, 0)),
                   pl.BlockSpec((BM, 1), lambda i: (i, 0))],
        out_shape=[jax.ShapeDtypeStruct((NPAD, 1), jnp.float32),
                   jax.ShapeDtypeStruct((NPAD, 1), jnp.float32)],
    )(m2, a0, a1, dg0, dg1, b2,
      wq1aT, bq1a, wq1bT, bq1b, wq2aT, bq2a, wq2bT, bq2b)


def kernel(obs, action, edge_index, W1, b1, W2, b2,
           Wq1a, bq1a, Wq1b, bq1b, Wq2a, bq2a, Wq2b, bq2b):
    src = edge_index[0].astype(jnp.int32)
    dst = edge_index[1].astype(jnp.int32)
    pad_idx = jnp.full((EPAD - E,), NPAD - 1, jnp.int32)
    src_pad = jnp.concatenate([src, pad_idx])
    dst_pad = jnp.concatenate([dst, pad_idx])

    x = jnp.concatenate([obs, action], axis=1)
    x_pad = jnp.pad(x, ((0, NPAD - N), (0, 0)))

    degp = _sc_degree(dst_pad)
    dg0, dg1 = degp[0], degp[1]

    h1 = _tc_matmul(x_pad, W1.T)
    m1 = _tc_scale(h1, dg0, dg1)
    acc1 = _sc_aggregate(m1, src_pad, dst_pad)

    m2 = _tc_mid(m1, acc1[0], acc1[1], dg0, dg1, b1.reshape(1, D), W2.T)
    acc2 = _sc_aggregate(m2, src_pad, dst_pad)

    q1p, q2p = _tc_final(
        m2, acc2[0], acc2[1], dg0, dg1, b2.reshape(1, D),
        Wq1a.T, bq1a.reshape(1, D), Wq1b.T, bq1b.reshape(1, 1),
        Wq2a.T, bq2a.reshape(1, D), Wq2b.T, bq2b.reshape(1, 1))
    return q1p[:N], q2p[:N]
